# Initial kernel scaffold; baseline (speedup 1.0000x reference)
import jax
import jax.numpy as jnp
from jax.experimental import pallas as pl


def kernel(x, base_W, base_b, gate_W, lora_A_W, lora_B_W):
    raise NotImplementedError("write your pallas kernel here")


# speedup vs baseline: 1.3963x; 1.3963x over previous
import jax
import jax.numpy as jnp
from jax.experimental import pallas as pl

_SCALING = 16.0 / 8.0


def _fused_kernel(x_ref, w_ref, b_ref, g_ref, a_ref, bb_ref, o_ref):
    xt = x_ref[...]
    base = jax.lax.dot_general(
        xt, w_ref[...], (((1,), (1,)), ((), ())),
        preferred_element_type=jnp.float32)
    logits = jax.lax.dot_general(
        xt, g_ref[...], (((1,), (1,)), ((), ())),
        preferred_element_type=jnp.float32)
    l0 = logits[:, 0:1]
    l1 = logits[:, 1:2]
    w = jnp.where(l0 >= l1, jax.nn.sigmoid(l0 - l1), 0.0) * _SCALING
    xa = jax.lax.dot_general(
        xt, a_ref[...], (((1,), (1,)), ((), ())),
        preferred_element_type=jnp.float32)
    xa = xa * w
    lora = jax.lax.dot_general(
        xa, bb_ref[...], (((1,), (1,)), ((), ())),
        preferred_element_type=jnp.float32)
    o_ref[...] = base + b_ref[...] + lora


def kernel(x, base_W, base_b, gate_W, lora_A_W, lora_B_W):
    n_tokens, in_f = x.shape
    out_f = base_W.shape[0]
    tm = 512
    grid = (n_tokens // tm,)
    bias2d = base_b.reshape(1, out_f)
    return pl.pallas_call(
        _fused_kernel,
        grid=grid,
        in_specs=[
            pl.BlockSpec((tm, in_f), lambda i: (i, 0)),
            pl.BlockSpec((out_f, in_f), lambda i: (0, 0)),
            pl.BlockSpec((1, out_f), lambda i: (0, 0)),
            pl.BlockSpec(gate_W.shape, lambda i: (0, 0)),
            pl.BlockSpec(lora_A_W.shape, lambda i: (0, 0)),
            pl.BlockSpec(lora_B_W.shape, lambda i: (0, 0)),
        ],
        out_specs=pl.BlockSpec((tm, out_f), lambda i: (i, 0)),
        out_shape=jax.ShapeDtypeStruct((n_tokens, out_f), jnp.float32),
    )(x, base_W, bias2d, gate_W, lora_A_W, lora_B_W)
